# baseline (speedup 1.0000x reference)
import functools

import jax
import jax.numpy as jnp
from jax import lax
from jax.experimental import pallas as pl
from jax.experimental.pallas import tpu as pltpu
from jax.experimental.pallas import tpu_sc as plsc

B = 16384
EMB = 16

_NC = 2
_NS = 16
_NW = _NC * _NS
_BPW = B // _NW
_CHUNK = 128
_NCHUNK = _BPW // _CHUNK


def _sc_gather_body(uidx_hbm, vidx_hbm, w_hbm, h_hbm, u_out, v_out,
                    uidx_v, vidx_v, urows, vrows, semu, semv):
    wid = lax.axis_index("s") * _NC + lax.axis_index("c")
    base = wid * _BPW
    row0 = wid * _NCHUNK
    cu = pltpu.async_copy(uidx_hbm.at[pl.ds(row0, _NCHUNK)], uidx_v, semu)
    cv = pltpu.async_copy(vidx_hbm.at[pl.ds(row0, _NCHUNK)], vidx_v, semv)
    cu.wait()
    gu = [pltpu.async_copy(w_hbm.at[uidx_v.at[j]],
                           urows.at[pl.ds(j * _CHUNK, _CHUNK)], semu)
          for j in range(_NCHUNK)]
    cv.wait()
    gv = [pltpu.async_copy(h_hbm.at[vidx_v.at[j]],
                           vrows.at[pl.ds(j * _CHUNK, _CHUNK)], semv)
          for j in range(_NCHUNK)]
    for g in gu:
        g.wait()
    ou = pltpu.async_copy(urows, u_out.at[pl.ds(base, _BPW)], semu)
    for g in gv:
        g.wait()
    ov = pltpu.async_copy(vrows, v_out.at[pl.ds(base, _BPW)], semv)
    ou.wait()
    ov.wait()


_sc_gather = functools.partial(
    pl.kernel,
    mesh=plsc.VectorSubcoreMesh(core_axis_name="c", subcore_axis_name="s"),
    out_type=[jax.ShapeDtypeStruct((B, EMB), jnp.float32),
              jax.ShapeDtypeStruct((B, EMB), jnp.float32)],
    scratch_types=[pltpu.VMEM((_NCHUNK, _CHUNK), jnp.int32),
                   pltpu.VMEM((_NCHUNK, _CHUNK), jnp.int32),
                   pltpu.VMEM((_BPW, EMB), jnp.float32),
                   pltpu.VMEM((_BPW, EMB), jnp.float32),
                   pltpu.SemaphoreType.DMA,
                   pltpu.SemaphoreType.DMA],
    compiler_params=pltpu.CompilerParams(use_tc_tiling_on_sc=False),
)(_sc_gather_body)


_BLK = 2048


def _mlp_body(u_ref, v_ref, w1_ref, w2_ref, b2_ref, o_ref):
    u = u_ref[...]
    v = v_ref[...]
    w1 = w1_ref[...]
    h = lax.dot_general(u, w1[:, :EMB], (((1,), (1,)), ((), ())),
                        preferred_element_type=jnp.float32)
    h = h + lax.dot_general(v, w1[:, EMB:], (((1,), (1,)), ((), ())),
                            preferred_element_type=jnp.float32)
    h = jnp.maximum(h, 0.0)
    logit = jnp.sum(h * w2_ref[...], axis=1) + b2_ref[0]
    o_ref[...] = 1.0 / (1.0 + jnp.exp(-logit))


def _mlp(u, v, w1, w2, b2):
    return pl.pallas_call(
        _mlp_body,
        grid=(B // _BLK,),
        in_specs=[
            pl.BlockSpec((_BLK, EMB), lambda i: (i, 0)),
            pl.BlockSpec((_BLK, EMB), lambda i: (i, 0)),
            pl.BlockSpec((EMB, 2 * EMB), lambda i: (0, 0)),
            pl.BlockSpec((1, EMB), lambda i: (0, 0)),
            pl.BlockSpec(memory_space=pltpu.SMEM),
        ],
        out_specs=pl.BlockSpec((_BLK,), lambda i: (i,)),
        out_shape=jax.ShapeDtypeStruct((B,), jnp.float32),
    )(u, v, w1, w2, b2)


def kernel(x, W, H, W1, W2, b2):
    uidx = x[:, 0].reshape(_NW * _NCHUNK, _CHUNK)
    vidx = x[:, 1].reshape(_NW * _NCHUNK, _CHUNK)
    U, V = _sc_gather(uidx, vidx, W, H)
    return _mlp(U, V, W1, W2, b2)
